# baseline (speedup 1.0000x reference)
import functools

import jax
import jax.numpy as jnp
from jax import lax
from jax.experimental import pallas as pl
from jax.experimental.pallas import tpu as pltpu
from jax.experimental.pallas import tpu_sc as plsc

D_MODEL = 512
TAB_ROWS = 24
B, L = 4096, 200
N = B * L
NC, NS = 2, 16
NW = NC * NS
PER_W = N // NW
CHUNK = 40
N_CHUNKS = PER_W // CHUNK
NB = 4
LANES = 16

_mesh = plsc.VectorSubcoreMesh(core_axis_name="c", subcore_axis_name="s")


@functools.partial(
    pl.kernel,
    out_type=jax.ShapeDtypeStruct((N, D_MODEL), jnp.float32),
    mesh=_mesh,
    compiler_params=pltpu.CompilerParams(
        use_tc_tiling_on_sc=False, needs_layout_passes=False
    ),
    scratch_types=[
        pltpu.VMEM((PER_W,), jnp.int32),
        pltpu.VMEM((TAB_ROWS * D_MODEL,), jnp.float32),
        pltpu.VMEM((NB, CHUNK, D_MODEL), jnp.float32),
        pltpu.SemaphoreType.DMA,
        pltpu.SemaphoreType.DMA,
        pltpu.SemaphoreType.DMA,
        pltpu.SemaphoreType.DMA,
    ],
)
def _embed_sc(idx_hbm, table_hbm, out_hbm, idx_v, table_v, stage_v, *osems):
    wid = lax.axis_index("s") * NC + lax.axis_index("c")
    base = wid * PER_W
    pltpu.sync_copy(table_hbm, table_v)
    pltpu.sync_copy(idx_hbm.at[pl.ds(base, PER_W)], idx_v)
    colv = lax.iota(jnp.int32, LANES)

    def o_dst(g):
        return out_hbm.at[pl.ds(base + g * CHUNK, CHUNK)]

    @pl.loop(0, N_CHUNKS, step=NB)
    def _outer(gg):
        for b in range(NB):
            g = gg + b

            @pl.when(g >= NB)
            def _():
                pltpu.make_async_copy(stage_v.at[b], o_dst(g - NB), osems[b]).wait()

            @plsc.parallel_loop(0, CHUNK, unroll=4)
            def _row(r):
                pos = jnp.full((LANES,), g * CHUNK + r, jnp.int32)
                rb = plsc.load_gather(idx_v, [pos]) * D_MODEL + colv
                for d in range(D_MODEL // LANES):
                    blk = table_v.at[pl.ds(d * LANES, TAB_ROWS * D_MODEL - d * LANES)]
                    vals = plsc.load_gather(blk, [rb])
                    stage_v[b, r, pl.ds(d * LANES, LANES)] = vals

            pltpu.async_copy(stage_v.at[b], o_dst(g), osems[b])

    for b in range(NB):
        pltpu.make_async_copy(
            stage_v.at[b], o_dst(N_CHUNKS - NB + b), osems[b]
        ).wait()


def kernel(x, holiday_embed):
    idx = x[:, :, -1].reshape(N)
    out = _embed_sc(idx, holiday_embed.reshape(TAB_ROWS * D_MODEL))
    return out.reshape(B, L, D_MODEL)


# speedup vs baseline: 1.0844x; 1.0844x over previous
import functools

import jax
import jax.numpy as jnp
from jax import lax
from jax.experimental import pallas as pl
from jax.experimental.pallas import tpu as pltpu
from jax.experimental.pallas import tpu_sc as plsc

D_MODEL = 512
TAB_ROWS = 24
B, L = 4096, 200
N = B * L
NC, NS = 2, 16
NW = NC * NS
PER_W = N // NW
CHUNK = 80
N_CHUNKS = PER_W // CHUNK
LANES = 16

_mesh = plsc.VectorSubcoreMesh(core_axis_name="c", subcore_axis_name="s")


@functools.partial(
    pl.kernel,
    out_type=jax.ShapeDtypeStruct((N, D_MODEL), jnp.float32),
    mesh=_mesh,
    compiler_params=pltpu.CompilerParams(
        use_tc_tiling_on_sc=False, needs_layout_passes=False
    ),
    scratch_types=[
        pltpu.VMEM((PER_W,), jnp.int32),
        pltpu.VMEM((TAB_ROWS * D_MODEL,), jnp.float32),
        pltpu.VMEM((2, CHUNK, D_MODEL), jnp.float32),
        pltpu.SemaphoreType.DMA,
        pltpu.SemaphoreType.DMA,
    ],
)
def _embed_sc(idx_hbm, table_hbm, out_hbm, idx_v, table_v, stage_v, osem0, osem1):
    osems = (osem0, osem1)
    wid = lax.axis_index("s") * NC + lax.axis_index("c")
    base = wid * PER_W
    pltpu.sync_copy(table_hbm, table_v)
    pltpu.sync_copy(idx_hbm.at[pl.ds(base, PER_W)], idx_v)
    colv = lax.iota(jnp.int32, LANES)

    def o_dst(g):
        return out_hbm.at[pl.ds(base + g * CHUNK, CHUNK)]

    @pl.loop(0, N_CHUNKS, step=2)
    def _outer(gg):
        for b in range(2):
            g = gg + b

            @pl.when(g > 1)
            def _():
                pltpu.make_async_copy(stage_v.at[b], o_dst(g - 2), osems[b]).wait()

            @plsc.parallel_loop(0, CHUNK, unroll=4)
            def _row(r):
                pos = jnp.full((LANES,), g * CHUNK + r, jnp.int32)
                rb = plsc.load_gather(idx_v, [pos]) * D_MODEL + colv
                for d in range(D_MODEL // LANES):
                    blk = table_v.at[pl.ds(d * LANES, TAB_ROWS * D_MODEL - d * LANES)]
                    vals = plsc.load_gather(blk, [rb])
                    stage_v[b, r, pl.ds(d * LANES, LANES)] = vals

            pltpu.async_copy(stage_v.at[b], o_dst(g), osems[b])

    pltpu.make_async_copy(stage_v.at[0], o_dst(N_CHUNKS - 2), osems[0]).wait()
    pltpu.make_async_copy(stage_v.at[1], o_dst(N_CHUNKS - 1), osems[1]).wait()


def kernel(x, holiday_embed):
    idx = x[:, :, -1].reshape(N)
    out = _embed_sc(idx, holiday_embed.reshape(TAB_ROWS * D_MODEL))
    return out.reshape(B, L, D_MODEL)
